# baseline (speedup 1.0000x reference)
import functools

import jax
import jax.numpy as jnp
from jax import lax
from jax.experimental import pallas as pl
from jax.experimental.pallas import tpu as pltpu
from jax.experimental.pallas import tpu_sc as plsc

B, H, W_IN, C = 2, 192, 192, 96
OH, OW = 2 * H, 2 * W_IN
FLAT_OUT = OH * OW * C
T = B * H * W_IN * C
OTOT = B * FLAT_OUT

NCHUNK = 16
CH = OTOT // NCHUNK
NPASS = NCHUNK // 2
PT = T // 16
WIN = 2304
KB = 4
NWIN = PT // WIN
NGRP = NWIN // KB
SL = CH // 16
ZB = 1024
DSTRIDE = CH // WIN

assert 16 * (2 * KB * WIN + ZB) + CH <= 2_097_151

_mesh = plsc.VectorSubcoreMesh(core_axis_name="c", subcore_axis_name="s")


@functools.partial(
    pl.kernel,
    out_type=jax.ShapeDtypeStruct((OTOT,), jnp.float32),
    mesh=_mesh,
    scratch_types=[
        [(pltpu.VMEM((WIN,), jnp.int32),
          pltpu.VMEM((WIN,), jnp.float32))
         for _ in range(KB)],
        pltpu.VMEM((ZB,), jnp.float32),
        pltpu.VMEM_SHARED((CH,), jnp.float32),
        pltpu.SemaphoreType.DMA,
    ],
)
def _scatter_kernel(idx_hbm, upd_hbm, out_hbm, bufs, zbuf, acc, sem):
    c = lax.axis_index("c")
    s = lax.axis_index("s")
    boff = jnp.where(s < 8, jnp.int32(0), jnp.int32(FLAT_OUT))
    lanes = lax.iota(jnp.int32, 16)
    zero16f = jnp.zeros((16,), jnp.float32)

    def init(i, carry):
        zbuf[pl.ds(i * 16, 16)] = zero16f
        return carry
    lax.fori_loop(0, ZB // 16, init, 0)

    def one_pass(p, carry):
        lo = (p * 2 + c) * CH

        def zero_slice(k, kcarry):
            pltpu.sync_copy(zbuf, acc.at[pl.ds(s * SL + k * ZB, ZB)])
            return kcarry
        lax.fori_loop(0, SL // ZB, zero_slice, 0)
        plsc.subcore_barrier()

        bl = boff - lo

        def one_group(g, gcarry):
            descs = []
            for b in range(KB):
                sidx, sval = bufs[b]
                base = s * PT + (g * KB + b) * WIN
                pltpu.sync_copy(idx_hbm.at[pl.ds(base, WIN)], sidx)
                pltpu.sync_copy(upd_hbm.at[pl.ds(base, WIN)], sval)

                def one_vec(i, vcarry):
                    u = sidx[pl.ds(i * 16, 16)] + bl
                    v = sval[pl.ds(i * 16, 16)]
                    m = (u >= 0) & (u < CH)
                    pos = i * 16 + lanes
                    sidx[pl.ds(i * 16, 16)] = jnp.where(m, u, pos * DSTRIDE)
                    sval[pl.ds(i * 16, 16)] = jnp.where(m, v,
                                                        jnp.float32(0.0))
                    return vcarry
                lax.fori_loop(0, WIN // 16, one_vec, 0)
                descs.append(pltpu.async_copy(sval, acc.at[sidx], sem,
                                              add=True))
            for d in descs:
                d.wait()
            return gcarry
        lax.fori_loop(0, NGRP, one_group, 0)
        plsc.subcore_barrier()
        pltpu.sync_copy(acc.at[pl.ds(s * SL, SL)],
                        out_hbm.at[pl.ds(lo + s * SL, SL)])
        return carry
    lax.fori_loop(0, NPASS, one_pass, 0)


def kernel(updates, mask):
    idx_flat = mask.reshape(-1).astype(jnp.int32)
    upd_flat = updates.reshape(-1)
    out = _scatter_kernel(idx_flat, upd_flat)
    return out.reshape(B, OH, OW, C)


# speedup vs baseline: 2.1605x; 2.1605x over previous
import functools

import jax
import jax.numpy as jnp
from jax import lax
from jax.experimental import pallas as pl
from jax.experimental.pallas import tpu as pltpu
from jax.experimental.pallas import tpu_sc as plsc

B, H, W_IN, C = 2, 192, 192, 96
OH, OW = 2 * H, 2 * W_IN
FLAT_OUT = OH * OW * C
T = B * H * W_IN * C
OTOT = B * FLAT_OUT

NCHUNK = 16
CH = OTOT // NCHUNK
NPASS = NCHUNK // 2
PT = T // 16
WIN = 9216
NWIN = PT // WIN
SL = CH // 16
ZB = 1024
DSTRIDE = CH // WIN

assert 16 * (2 * WIN + ZB) + CH <= 2_097_151

_mesh = plsc.VectorSubcoreMesh(core_axis_name="c", subcore_axis_name="s")


@functools.partial(
    pl.kernel,
    out_type=jax.ShapeDtypeStruct((OTOT,), jnp.float32),
    mesh=_mesh,
    scratch_types=[
        pltpu.VMEM((WIN,), jnp.int32),
        pltpu.VMEM((WIN,), jnp.float32),
        pltpu.VMEM((ZB,), jnp.float32),
        pltpu.VMEM_SHARED((CH,), jnp.float32),
    ],
)
def _scatter_kernel(idx_hbm, upd_hbm, out_hbm, sidx, sval, zbuf, acc):
    c = lax.axis_index("c")
    s = lax.axis_index("s")
    boff = jnp.where(s < 8, jnp.int32(0), jnp.int32(FLAT_OUT))
    lanes = lax.iota(jnp.int32, 16)
    zero16f = jnp.zeros((16,), jnp.float32)

    def init(i, carry):
        zbuf[pl.ds(i * 16, 16)] = zero16f
        return carry
    lax.fori_loop(0, ZB // 16, init, 0)

    def one_pass(p, carry):
        lo = (p * 2 + c) * CH

        def zero_slice(k, kcarry):
            pltpu.sync_copy(zbuf, acc.at[pl.ds(s * SL + k * ZB, ZB)])
            return kcarry
        lax.fori_loop(0, SL // ZB, zero_slice, 0)
        plsc.subcore_barrier()

        bl = boff - lo

        def one_window(w, wcarry):
            base = s * PT + w * WIN
            pltpu.sync_copy(idx_hbm.at[pl.ds(base, WIN)], sidx)
            pltpu.sync_copy(upd_hbm.at[pl.ds(base, WIN)], sval)

            def one_vec(i, vcarry):
                u = sidx[pl.ds(i * 16, 16)] + bl
                v = sval[pl.ds(i * 16, 16)]
                m = (u >= 0) & (u < CH)
                pos = i * 16 + lanes
                sidx[pl.ds(i * 16, 16)] = jnp.where(m, u, pos * DSTRIDE)
                sval[pl.ds(i * 16, 16)] = jnp.where(m, v, jnp.float32(0.0))
                return vcarry
            lax.fori_loop(0, WIN // 16, one_vec, 0)
            pltpu.sync_copy(sval, acc.at[sidx], add=True)
            return wcarry
        lax.fori_loop(0, NWIN, one_window, 0)
        plsc.subcore_barrier()
        pltpu.sync_copy(acc.at[pl.ds(s * SL, SL)],
                        out_hbm.at[pl.ds(lo + s * SL, SL)])
        return carry
    lax.fori_loop(0, NPASS, one_pass, 0)


def kernel(updates, mask):
    idx_flat = mask.reshape(-1).astype(jnp.int32)
    upd_flat = updates.reshape(-1)
    out = _scatter_kernel(idx_flat, upd_flat)
    return out.reshape(B, OH, OW, C)


# speedup vs baseline: 5.1090x; 2.3647x over previous
import functools

import jax
import jax.numpy as jnp
from jax import lax
from jax.experimental import pallas as pl
from jax.experimental.pallas import tpu as pltpu
from jax.experimental.pallas import tpu_sc as plsc

B, H, W_IN, C = 2, 192, 192, 96
OH, OW = 2 * H, 2 * W_IN
FLAT_OUT = OH * OW * C
T = B * H * W_IN * C
OTOT = B * FLAT_OUT

NCHUNK = 16
CH = OTOT // NCHUNK
NPASS = NCHUNK // 2
PT = T // 16
WIN = 8192
NWIN = PT // WIN
SL = CH // 16
ZB = 2048
DSTRIDE = CH // WIN

assert 16 * (2 * WIN + ZB) + CH <= 2_097_151

_mesh = plsc.VectorSubcoreMesh(core_axis_name="c", subcore_axis_name="s")


@functools.partial(
    pl.kernel,
    out_type=jax.ShapeDtypeStruct((OTOT,), jnp.float32),
    mesh=_mesh,
    scratch_types=[
        pltpu.VMEM((WIN,), jnp.int32),
        pltpu.VMEM((WIN,), jnp.float32),
        pltpu.VMEM((ZB,), jnp.float32),
        pltpu.VMEM_SHARED((CH,), jnp.float32),
    ],
)
def _scatter_kernel(idx_hbm, upd_hbm, out_hbm, sidx, sval, zbuf, acc):
    c = lax.axis_index("c")
    s = lax.axis_index("s")
    boff = jnp.where(s < 8, jnp.int32(0), jnp.int32(FLAT_OUT))
    lanes = lax.iota(jnp.int32, 16)
    zero16f = jnp.zeros((16,), jnp.float32)

    def init(i, carry):
        zbuf[pl.ds(i * 16, 16)] = zero16f
        return carry
    lax.fori_loop(0, ZB // 16, init, 0)

    def one_pass(p, carry):
        lo = (p * 2 + c) * CH

        def zero_slice(k, kcarry):
            pltpu.sync_copy(zbuf, acc.at[pl.ds(s * SL + k * ZB, ZB)])
            return kcarry
        lax.fori_loop(0, SL // ZB, zero_slice, 0)
        plsc.subcore_barrier()

        bl = boff - lo

        def one_window(w, wcarry):
            base = s * PT + w * WIN
            pltpu.sync_copy(idx_hbm.at[pl.ds(base, WIN)], sidx)
            pltpu.sync_copy(upd_hbm.at[pl.ds(base, WIN)], sval)

            def one_vec(i, vcarry):
                u = sidx[pl.ds(i * 16, 16)] + bl
                v = sval[pl.ds(i * 16, 16)]
                m = (u >= 0) & (u < CH)
                pos = i * 16 + lanes
                sidx[pl.ds(i * 16, 16)] = jnp.where(m, u, pos * DSTRIDE)
                sval[pl.ds(i * 16, 16)] = jnp.where(m, v, jnp.float32(0.0))
                return vcarry
            lax.fori_loop(0, WIN // 16, one_vec, 0)
            pltpu.sync_copy(sval, acc.at[sidx], add=True)
            return wcarry
        lax.fori_loop(0, NWIN, one_window, 0)
        plsc.subcore_barrier()
        pltpu.sync_copy(acc.at[pl.ds(s * SL, SL)],
                        out_hbm.at[pl.ds(lo + s * SL, SL)])
        return carry
    lax.fori_loop(0, NPASS, one_pass, 0)


def kernel(updates, mask):
    idx_flat = mask.reshape(-1).astype(jnp.int32)
    upd_flat = updates.reshape(-1)
    out = _scatter_kernel(idx_flat, upd_flat)
    return out.reshape(B, OH, OW, C)


# speedup vs baseline: 5.8507x; 1.1452x over previous
import functools

import jax
import jax.numpy as jnp
from jax import lax
from jax.experimental import pallas as pl
from jax.experimental.pallas import tpu as pltpu
from jax.experimental.pallas import tpu_sc as plsc

B, H, W_IN, C = 2, 192, 192, 96
OH, OW = 2 * H, 2 * W_IN
FLAT_OUT = OH * OW * C
T = B * H * W_IN * C
OTOT = B * FLAT_OUT

NCHUNK = 16
CH = OTOT // NCHUNK
NPASS = NCHUNK // 2
PT = T // 16
WIN = 8192
NWIN = PT // WIN
SL = CH // 16
ZB = 2048
DSTRIDE = CH // WIN

assert 16 * (2 * WIN + ZB) + CH <= 2_097_151

_mesh = plsc.VectorSubcoreMesh(core_axis_name="c", subcore_axis_name="s")


@functools.partial(
    pl.kernel,
    out_type=jax.ShapeDtypeStruct((OTOT,), jnp.float32),
    mesh=_mesh,
    scratch_types=[
        pltpu.VMEM((WIN,), jnp.int32),
        pltpu.VMEM((WIN,), jnp.float32),
        pltpu.VMEM((ZB,), jnp.float32),
        pltpu.VMEM_SHARED((CH,), jnp.float32),
        pltpu.SemaphoreType.DMA,
        pltpu.SemaphoreType.DMA,
    ],
)
def _scatter_kernel(idx_hbm, upd_hbm, out_hbm, sidx, sval, zbuf, acc,
                    semi, semv):
    c = lax.axis_index("c")
    s = lax.axis_index("s")
    boff = jnp.where(s < 8, jnp.int32(0), jnp.int32(FLAT_OUT))
    lanes = lax.iota(jnp.int32, 16)
    zero16f = jnp.zeros((16,), jnp.float32)

    def init(i, carry):
        zbuf[pl.ds(i * 16, 16)] = zero16f
        return carry
    lax.fori_loop(0, ZB // 16, init, 0)

    def one_pass(p, carry):
        lo = (p * 2 + c) * CH

        def zero_slice(k, kcarry):
            pltpu.sync_copy(zbuf, acc.at[pl.ds(s * SL + k * ZB, ZB)])
            return kcarry
        lax.fori_loop(0, SL // ZB, zero_slice, 0)
        plsc.subcore_barrier()

        bl = boff - lo

        dummy0 = lanes * DSTRIDE
        dstep = jnp.full((16,), 16 * DSTRIDE, jnp.int32)

        def one_window(w, wcarry):
            base = s * PT + w * WIN
            di = pltpu.async_copy(idx_hbm.at[pl.ds(base, WIN)], sidx, semi)
            dv = pltpu.async_copy(upd_hbm.at[pl.ds(base, WIN)], sval, semv)
            di.wait()
            dv.wait()

            def one_vec(i, dummy):
                u = sidx[pl.ds(i * 16, 16)] + bl
                v = sval[pl.ds(i * 16, 16)]
                m = lax.bitcast_convert_type(u, jnp.uint32) < jnp.uint32(CH)
                sidx[pl.ds(i * 16, 16)] = jnp.where(m, u, dummy)
                sval[pl.ds(i * 16, 16)] = jnp.where(m, v, jnp.float32(0.0))
                return dummy + dstep
            lax.fori_loop(0, WIN // 16, one_vec, dummy0)
            pltpu.sync_copy(sval, acc.at[sidx], add=True)
            return wcarry
        lax.fori_loop(0, NWIN, one_window, 0)
        plsc.subcore_barrier()
        pltpu.sync_copy(acc.at[pl.ds(s * SL, SL)],
                        out_hbm.at[pl.ds(lo + s * SL, SL)])
        return carry
    lax.fori_loop(0, NPASS, one_pass, 0)


def kernel(updates, mask):
    idx_flat = mask.reshape(-1).astype(jnp.int32)
    upd_flat = updates.reshape(-1)
    out = _scatter_kernel(idx_flat, upd_flat)
    return out.reshape(B, OH, OW, C)
